# Initial kernel scaffold; baseline (speedup 1.0000x reference)
import functools

import jax
import jax.numpy as jnp
from jax import lax
from jax.experimental import pallas as pl
from jax.experimental.pallas import tpu as pltpu
from jax.experimental.pallas import tpu_sc as plsc

_BN_EPS = 1e-5

_NC = 2
_NS = 16
_NW = _NC * _NS
_B = 128


def _proj_body(x_ref, c_ref, w1x_ref, w1c_ref, hp_ref):
    hp_ref[...] = (
        jnp.dot(x_ref[...], w1x_ref[...].T, preferred_element_type=jnp.float32)
        + jnp.dot(c_ref[...], w1c_ref[...].T, preferred_element_type=jnp.float32)
    )


def _project(x, c, W1):
    n = x.shape[0]
    d_hid = W1.shape[0]
    w1x = W1[:, : x.shape[1]]
    w1c = W1[:, x.shape[1] :]
    return pl.pallas_call(
        _proj_body,
        out_shape=jax.ShapeDtypeStruct((n, d_hid), jnp.float32),
    )(x, c, w1x, w1c)


def _sc_body(nb, rows_per_tile, hp_hbm, srcs_hbm, dsts_hbm, zer_hbm, out_hbm,
             src_v, dst_v, rows_v, acc_sh, sem):
    cid = lax.axis_index("c")
    sid = lax.axis_index("s")
    w = cid * _NS + sid
    base = sid * rows_per_tile
    pltpu.sync_copy(zer_hbm, acc_sh.at[pl.ds(base, rows_per_tile)])
    pltpu.sync_copy(srcs_hbm.at[w], src_v)
    pltpu.sync_copy(dsts_hbm.at[w], dst_v)
    plsc.subcore_barrier()

    @pl.loop(0, nb)
    def _batch(b):
        pltpu.async_copy(hp_hbm.at[src_v.at[b]], rows_v, sem).wait()
        pltpu.sync_copy(rows_v, acc_sh.at[dst_v.at[b]], add=True)

    plsc.subcore_barrier()
    pltpu.sync_copy(
        acc_sh.at[pl.ds(base, rows_per_tile)],
        out_hbm.at[cid, pl.ds(base, rows_per_tile)],
    )


def _sc_segment_sum(hp, src, dst, n_pad):
    e = src.shape[0]
    d = hp.shape[1]
    per_w = -(-e // _NW)
    per_w_pad = -(-per_w // _B) * _B
    nb = per_w_pad // _B
    e_pad = per_w_pad * _NW
    rows_per_tile = n_pad // _NS

    src_p = jnp.zeros((e_pad,), jnp.int32).at[:e].set(src.astype(jnp.int32))
    dst_p = jnp.full((e_pad,), n_pad - 1, jnp.int32).at[:e].set(
        dst.astype(jnp.int32)
    )
    srcs = src_p.reshape(_NW, nb, _B)
    dsts = dst_p.reshape(_NW, nb, _B)
    zer = jnp.zeros((rows_per_tile, d), jnp.float32)

    mesh = plsc.VectorSubcoreMesh(core_axis_name="c", subcore_axis_name="s")
    fn = pl.kernel(
        functools.partial(_sc_body, nb, rows_per_tile),
        out_type=jax.ShapeDtypeStruct((_NC, n_pad, d), jnp.float32),
        mesh=mesh,
        scratch_types=[
            pltpu.VMEM((nb, _B), jnp.int32),
            pltpu.VMEM((nb, _B), jnp.int32),
            pltpu.VMEM((_B, d), jnp.float32),
            pltpu.VMEM_SHARED((n_pad, d), jnp.float32),
            pltpu.SemaphoreType.DMA,
        ],
    )
    return fn(hp, srcs, dsts, zer)


def _mlp_body(n, hp_ref, agg_ref, b1_ref, gamma_ref, beta_ref, w2_ref, b2_ref,
              eps_ref, out_ref):
    hp = hp_ref[...]
    y = (
        (1.0 + eps_ref[0, 0]) * hp
        + agg_ref[0, :n, :]
        + agg_ref[1, :n, :]
        + b1_ref[...]
    )
    mu = jnp.mean(y, axis=0, keepdims=True)
    var = jnp.mean(jnp.square(y - mu), axis=0, keepdims=True)
    yhat = (y - mu) * lax.rsqrt(var + _BN_EPS)
    y2 = jnp.maximum(yhat * gamma_ref[...] + beta_ref[...], 0.0)
    out_ref[...] = (
        jnp.dot(y2, w2_ref[...].T, preferred_element_type=jnp.float32)
        + b2_ref[...]
    )


def _mlp(hp, agg, b1, gamma, beta, W2, b2, eps):
    n, d_hid = hp.shape
    emb = W2.shape[0]
    return pl.pallas_call(
        functools.partial(_mlp_body, n),
        out_shape=jax.ShapeDtypeStruct((n, emb), jnp.float32),
        in_specs=[pl.BlockSpec(memory_space=pltpu.VMEM) for _ in range(7)]
        + [pl.BlockSpec(memory_space=pltpu.SMEM)],
        out_specs=pl.BlockSpec(memory_space=pltpu.VMEM),
    )(
        hp,
        agg,
        b1.reshape(1, d_hid),
        gamma.reshape(1, d_hid),
        beta.reshape(1, d_hid),
        W2,
        b2.reshape(1, emb),
        eps.reshape(1, 1),
    )


def kernel(x, c, edge_index, W1, b1, gamma, beta, W2, b2, eps):
    n = x.shape[0]
    n_pad = -(-(n + 1) // _NS) * _NS
    hp = _project(x, c, W1)
    agg = _sc_segment_sum(hp, edge_index[0], edge_index[1], n_pad)
    return _mlp(hp, agg, b1, gamma, beta, W2, b2, eps)


# speedup vs baseline: 6.2836x; 6.2836x over previous
import functools

import jax
import jax.numpy as jnp
from jax import lax
from jax.experimental import pallas as pl
from jax.experimental.pallas import tpu as pltpu
from jax.experimental.pallas import tpu_sc as plsc

_BN_EPS = 1e-5

_NC = 2
_NS = 16
_NW = _NC * _NS
_B = 128


def _proj_body(x_ref, c_ref, w1x_ref, w1c_ref, hp_ref):
    hp_ref[...] = (
        jnp.dot(x_ref[...], w1x_ref[...].T, preferred_element_type=jnp.float32)
        + jnp.dot(c_ref[...], w1c_ref[...].T, preferred_element_type=jnp.float32)
    )


def _project(x, c, W1):
    n = x.shape[0]
    d_hid = W1.shape[0]
    w1x = W1[:, : x.shape[1]]
    w1c = W1[:, x.shape[1] :]
    return pl.pallas_call(
        _proj_body,
        out_shape=jax.ShapeDtypeStruct((n, d_hid), jnp.float32),
    )(x, c, w1x, w1c)


def _sc_body(nb, rows_per_tile, hp_hbm, srcs_hbm, dsts_hbm, zer_hbm, out_hbm,
             src_v, dst_v, rows_v, acc_sh, sem):
    cid = lax.axis_index("c")
    sid = lax.axis_index("s")
    w = cid * _NS + sid
    base = sid * rows_per_tile
    pltpu.sync_copy(zer_hbm, acc_sh.at[pl.ds(base, rows_per_tile)])
    pltpu.sync_copy(srcs_hbm.at[w], src_v)
    pltpu.sync_copy(dsts_hbm.at[w], dst_v)
    plsc.subcore_barrier()

    @pl.loop(0, nb)
    def _batch(b):
        pltpu.async_copy(hp_hbm.at[src_v.at[b]], rows_v, sem).wait()
        pltpu.sync_copy(rows_v, acc_sh.at[dst_v.at[b]], add=True)

    plsc.subcore_barrier()
    pltpu.sync_copy(
        acc_sh.at[pl.ds(base, rows_per_tile)],
        out_hbm.at[cid, pl.ds(base, rows_per_tile)],
    )


def _sc_segment_sum(hp, src, dst, n_pad):
    e = src.shape[0]
    d = hp.shape[1]
    per_w = -(-e // _NW)
    per_w_pad = -(-per_w // _B) * _B
    nb = per_w_pad // _B
    e_pad = per_w_pad * _NW
    rows_per_tile = n_pad // _NS

    src_p = jnp.zeros((e_pad,), jnp.int32).at[:e].set(src.astype(jnp.int32))
    dst_p = jnp.full((e_pad,), n_pad - 1, jnp.int32).at[:e].set(
        dst.astype(jnp.int32)
    )
    srcs = src_p.reshape(_NW, nb, _B)
    dsts = dst_p.reshape(_NW, nb, _B)
    zer = jnp.zeros((rows_per_tile, d), jnp.float32)

    mesh = plsc.VectorSubcoreMesh(
        core_axis_name="c", subcore_axis_name="s", num_cores=_NC,
        num_subcores=_NS,
    )
    fn = pl.kernel(
        functools.partial(_sc_body, nb, rows_per_tile),
        out_type=jax.ShapeDtypeStruct((_NC, n_pad, d), jnp.float32),
        mesh=mesh,
        scratch_types=[
            pltpu.VMEM((nb, _B), jnp.int32),
            pltpu.VMEM((nb, _B), jnp.int32),
            pltpu.VMEM((_B, d), jnp.float32),
            pltpu.VMEM_SHARED((n_pad, d), jnp.float32),
            pltpu.SemaphoreType.DMA,
        ],
    )
    return fn(hp, srcs, dsts, zer)


def _mlp_body(n, hp_ref, agg_ref, b1_ref, gamma_ref, beta_ref, w2_ref, b2_ref,
              eps_ref, out_ref):
    hp = hp_ref[...]
    y = (
        (1.0 + eps_ref[0, 0]) * hp
        + agg_ref[0, :n, :]
        + agg_ref[1, :n, :]
        + b1_ref[...]
    )
    mu = jnp.mean(y, axis=0, keepdims=True)
    var = jnp.mean(jnp.square(y - mu), axis=0, keepdims=True)
    yhat = (y - mu) * lax.rsqrt(var + _BN_EPS)
    y2 = jnp.maximum(yhat * gamma_ref[...] + beta_ref[...], 0.0)
    out_ref[...] = (
        jnp.dot(y2, w2_ref[...].T, preferred_element_type=jnp.float32)
        + b2_ref[...]
    )


def _mlp(hp, agg, b1, gamma, beta, W2, b2, eps):
    n, d_hid = hp.shape
    emb = W2.shape[0]
    return pl.pallas_call(
        functools.partial(_mlp_body, n),
        out_shape=jax.ShapeDtypeStruct((n, emb), jnp.float32),
        in_specs=[pl.BlockSpec(memory_space=pltpu.VMEM) for _ in range(7)]
        + [pl.BlockSpec(memory_space=pltpu.SMEM)],
        out_specs=pl.BlockSpec(memory_space=pltpu.VMEM),
    )(
        hp,
        agg,
        b1.reshape(1, d_hid),
        gamma.reshape(1, d_hid),
        beta.reshape(1, d_hid),
        W2,
        b2.reshape(1, emb),
        eps.reshape(1, 1),
    )


def kernel(x, c, edge_index, W1, b1, gamma, beta, W2, b2, eps):
    n = x.shape[0]
    n_pad = -(-(n + 1) // (_NS * 8)) * (_NS * 8)
    hp = _project(x, c, W1)
    agg = _sc_segment_sum(hp, edge_index[0], edge_index[1], n_pad)
    return _mlp(hp, agg, b1, gamma, beta, W2, b2, eps)
